# baseline (speedup 1.0000x reference)
import jax
import jax.numpy as jnp
from jax import lax
from jax.experimental import pallas as pl
from jax.experimental.pallas import tpu as pltpu
from jax.experimental.pallas import tpu_sc as plsc

NC, NS, LANES = 2, 16, 16
NW = NC * NS
SEQ = 128
D = 64
V = 1000000
B = 4096
DV = D // LANES
PW1 = 72
PW2 = 136

TC_ = 256
NCHUNK1 = V // TC_
VTAIL = V - NCHUNK1 * TC_
ITER1 = (NCHUNK1 + NW - 1) // NW
NBUF1 = 4
NWB = 2
CHW = TC_ * PW1

SEQ_PER_W = B // NW
SPC = 2
NCHUNK2 = SEQ_PER_W // SPC
NBUF2 = 4


def _wid():
    return lax.axis_index("s") * NC + lax.axis_index("c")


def _transpose_kernel(tblt_hbm, tail_hbm, packed_hbm, strip_v, tail_v,
                      packed_v, rsem0, rsem1, rsem2, rsem3, wsem0, wsem1):
    wid = _wid()
    rsems = (rsem0, rsem1, rsem2, rsem3)
    wsems = (wsem0, wsem1)
    lanep = lax.iota(jnp.int32, LANES) * PW1

    def fire(i, buf):
        k = wid + i * NW

        @pl.when(k < NCHUNK1)
        def _():
            v0 = pl.multiple_of(k * TC_, TC_)
            pltpu.async_copy(
                tblt_hbm.at[:, pl.ds(v0, TC_)], strip_v.at[buf], rsems[buf]
            )

    def transpose(src, dst, width):
        @plsc.parallel_loop(0, width // LANES, unroll=8)
        def _(vg):
            base = vg * (LANES * PW1)
            for d in range(D):
                v = src[d, pl.ds(vg * LANES, LANES)]
                plsc.store_scatter(dst, [lanep + (base + d)], v)

    def process(i, buf, wbuf):
        k = wid + i * NW

        @pl.when(k < NCHUNK1)
        def _():
            pltpu.make_async_copy(
                tblt_hbm.at[:, pl.ds(0, TC_)], strip_v.at[buf], rsems[buf]
            ).wait()
            dst = packed_v.at[pl.ds(wbuf * CHW, CHW)]
            @pl.when(i >= NWB)
            def _():
                pltpu.make_async_copy(
                    dst, packed_hbm.at[pl.ds(0, CHW)], wsems[wbuf]
                ).wait()
            transpose(strip_v.at[buf], dst, TC_)
            start = pl.multiple_of(k * CHW, CHW)
            pltpu.async_copy(
                dst, packed_hbm.at[pl.ds(start, CHW)], wsems[wbuf]
            )

    for p in range(NBUF1 - 1):
        fire(p, p)

    @pl.loop(0, ITER1 + (NBUF1 - ITER1 % NBUF1) % NBUF1, step=NBUF1)
    def _chunks(i0):
        for b in range(NBUF1):
            i = i0 + b
            fire(i + NBUF1 - 1, (b + NBUF1 - 1) % NBUF1)
            process(i, b, b % NWB)

    for wbuf in range(NWB):
        pltpu.make_async_copy(
            packed_v.at[pl.ds(wbuf * CHW, CHW)],
            packed_hbm.at[pl.ds(0, CHW)],
            wsems[wbuf],
        ).wait()

    @pl.when(wid == 0)
    def _():
        pltpu.sync_copy(tail_hbm, tail_v)
        dst = packed_v.at[pl.ds(0, VTAIL * PW1)]
        transpose(tail_v, dst, VTAIL)
        start = pl.multiple_of(NCHUNK1 * CHW, PW1)
        pltpu.sync_copy(dst, packed_hbm.at[pl.ds(start, VTAIL * PW1)])


def _gather_kernel(x_hbm, packed_hbm, pos_hbm, out_hbm, idx_v, pos_v, rows_v,
                   outt_v, sem0, sem1, sem2, sem3):
    wid = _wid()
    seq_base = pl.multiple_of(wid * SEQ_PER_W, SEQ_PER_W)
    pltpu.sync_copy(x_hbm.at[pl.ds(seq_base, SEQ_PER_W)], idx_v)
    pltpu.sync_copy(pos_hbm, pos_v)

    sems = (sem0, sem1, sem2, sem3)
    lane1 = lax.iota(jnp.int32, LANES)
    zerov = lane1 * 0

    def fire(g, buf):
        @pl.when(g < NCHUNK2)
        def _():
            for s in range(SPC):
                pltpu.async_copy(
                    packed_hbm.at[idx_v.at[g * SPC + s]],
                    rows_v.at[buf, pl.ds(s * SEQ, SEQ)],
                    sems[buf],
                )

    def drain(buf):
        pltpu.make_async_copy(
            packed_hbm.at[pl.ds(0, SPC * SEQ)], rows_v.at[buf], sems[buf]
        ).wait()

    def process(g, buf):
        @plsc.parallel_loop(0, SEQ, unroll=16)
        def _(l):
            idx_l = zerov + l
            for c in range(DV):
                vp = pos_v[l, pl.ds(c * LANES, LANES)]
                idx_d = lane1 + c * LANES
                for s in range(SPC):
                    v = rows_v[buf, s * SEQ + l, pl.ds(c * LANES, LANES)]
                    plsc.store_scatter(outt_v.at[s], [idx_d, idx_l], v + vp)
        for s in range(SPC):
            pltpu.sync_copy(
                outt_v.at[s, :, pl.ds(0, SEQ)],
                out_hbm.at[seq_base + g * SPC + s],
            )

    for p in range(NBUF2 - 1):
        fire(p, p)

    @pl.loop(0, NCHUNK2, step=NBUF2)
    def _chunks(g0):
        for b in range(NBUF2):
            g = g0 + b
            fire(g + NBUF2 - 1, (b + NBUF2 - 1) % NBUF2)
            drain(b)
            process(g, b)


def kernel(x, table, pos):
    mesh = plsc.VectorSubcoreMesh(
        core_axis_name="c", subcore_axis_name="s",
        num_cores=NC, num_subcores=NS,
    )
    tblt = jnp.swapaxes(table, 0, 1)
    tail = jnp.swapaxes(table[NCHUNK1 * TC_:, :], 0, 1)

    packed = pl.kernel(
        _transpose_kernel,
        out_type=jax.ShapeDtypeStruct((V * PW1,), jnp.float32),
        mesh=mesh,
        scratch_types=[
            pltpu.VMEM((NBUF1, D, TC_), jnp.float32),
            pltpu.VMEM((D, VTAIL), jnp.float32),
            pltpu.VMEM((NWB * CHW,), jnp.float32),
        ] + [pltpu.SemaphoreType.DMA] * (NBUF1 + NWB),
        compiler_params=pltpu.CompilerParams(
            use_tc_tiling_on_sc=True, needs_layout_passes=False,
        ),
    )(tblt, tail)

    out = pl.kernel(
        _gather_kernel,
        out_type=jax.ShapeDtypeStruct((B, D, SEQ), jnp.float32),
        mesh=mesh,
        scratch_types=[
            pltpu.VMEM((SEQ_PER_W, SEQ), jnp.int32),
            pltpu.VMEM((SEQ, D), jnp.float32),
            pltpu.VMEM((NBUF2, SPC * SEQ, PW1), jnp.float32),
            pltpu.VMEM((SPC, D, PW2), jnp.float32),
            pltpu.SemaphoreType.DMA,
            pltpu.SemaphoreType.DMA,
            pltpu.SemaphoreType.DMA,
            pltpu.SemaphoreType.DMA,
        ],
        compiler_params=pltpu.CompilerParams(
            use_tc_tiling_on_sc=False, needs_layout_passes=False,
        ),
    )(x, packed.reshape(V, PW1), pos)

    return jnp.swapaxes(out, 1, 2)


# speedup vs baseline: 1.0268x; 1.0268x over previous
import jax
import jax.numpy as jnp
from jax import lax
from jax.experimental import pallas as pl
from jax.experimental.pallas import tpu as pltpu
from jax.experimental.pallas import tpu_sc as plsc

NC, NS, LANES = 2, 16, 16
NW = NC * NS
SEQ = 128
D = 64
V = 1000000
B = 4096
DV = D // LANES
PW1 = 72
PW2 = 136

TC_ = 256
NCHUNK1 = V // TC_
VTAIL = V - NCHUNK1 * TC_
ITER1 = (NCHUNK1 + NW - 1) // NW
NBUF1 = 4
NWB = 2
CHW = TC_ * PW1

SEQ_PER_W = B // NW
SPC = 2
NCHUNK2 = SEQ_PER_W // SPC
NBUF2 = 4


def _wid():
    return lax.axis_index("s") * NC + lax.axis_index("c")


def _transpose_kernel(tblt_hbm, tail_hbm, packed_hbm, strip_v, tail_v,
                      packed_v, rsem0, rsem1, rsem2, rsem3, wsem0, wsem1):
    wid = _wid()
    rsems = (rsem0, rsem1, rsem2, rsem3)
    wsems = (wsem0, wsem1)
    lanep = lax.iota(jnp.int32, LANES) * PW1

    def fire(i, buf):
        k = wid + i * NW

        @pl.when(k < NCHUNK1)
        def _():
            v0 = pl.multiple_of(k * TC_, TC_)
            pltpu.async_copy(
                tblt_hbm.at[:, pl.ds(v0, TC_)], strip_v.at[buf], rsems[buf]
            )

    def transpose(src, dst, width):
        @plsc.parallel_loop(0, width // LANES, unroll=4)
        def _(vg):
            base = vg * (LANES * PW1)
            for d in range(D):
                v = src[d, pl.ds(vg * LANES, LANES)]
                plsc.store_scatter(dst, [lanep + (base + d)], v)

    def process(i, buf, wbuf):
        k = wid + i * NW

        @pl.when(k < NCHUNK1)
        def _():
            pltpu.make_async_copy(
                tblt_hbm.at[:, pl.ds(0, TC_)], strip_v.at[buf], rsems[buf]
            ).wait()
            dst = packed_v.at[pl.ds(wbuf * CHW, CHW)]
            @pl.when(i >= NWB)
            def _():
                pltpu.make_async_copy(
                    dst, packed_hbm.at[pl.ds(0, CHW)], wsems[wbuf]
                ).wait()
            transpose(strip_v.at[buf], dst, TC_)
            start = pl.multiple_of(k * CHW, CHW)
            pltpu.async_copy(
                dst, packed_hbm.at[pl.ds(start, CHW)], wsems[wbuf]
            )

    for p in range(NBUF1 - 1):
        fire(p, p)

    @pl.loop(0, ITER1 + (NBUF1 - ITER1 % NBUF1) % NBUF1, step=NBUF1)
    def _chunks(i0):
        for b in range(NBUF1):
            i = i0 + b
            fire(i + NBUF1 - 1, (b + NBUF1 - 1) % NBUF1)
            process(i, b, b % NWB)

    for wbuf in range(NWB):
        pltpu.make_async_copy(
            packed_v.at[pl.ds(wbuf * CHW, CHW)],
            packed_hbm.at[pl.ds(0, CHW)],
            wsems[wbuf],
        ).wait()

    @pl.when(wid == 0)
    def _():
        pltpu.sync_copy(tail_hbm, tail_v)
        dst = packed_v.at[pl.ds(0, VTAIL * PW1)]
        transpose(tail_v, dst, VTAIL)
        start = pl.multiple_of(NCHUNK1 * CHW, PW1)
        pltpu.sync_copy(dst, packed_hbm.at[pl.ds(start, VTAIL * PW1)])


def _gather_kernel(x_hbm, packed_hbm, pos_hbm, out_hbm, idx_v, pos_v, rows_v,
                   outt_v, sem0, sem1, sem2, sem3):
    wid = _wid()
    seq_base = pl.multiple_of(wid * SEQ_PER_W, SEQ_PER_W)
    pltpu.sync_copy(x_hbm.at[pl.ds(seq_base, SEQ_PER_W)], idx_v)
    pltpu.sync_copy(pos_hbm, pos_v)

    sems = (sem0, sem1, sem2, sem3)
    lane1 = lax.iota(jnp.int32, LANES)
    zerov = lane1 * 0

    def fire(g, buf):
        @pl.when(g < NCHUNK2)
        def _():
            for s in range(SPC):
                pltpu.async_copy(
                    packed_hbm.at[idx_v.at[g * SPC + s]],
                    rows_v.at[buf, pl.ds(s * SEQ, SEQ)],
                    sems[buf],
                )

    def drain(buf):
        pltpu.make_async_copy(
            packed_hbm.at[pl.ds(0, SPC * SEQ)], rows_v.at[buf], sems[buf]
        ).wait()

    def process(g, buf):
        @plsc.parallel_loop(0, SEQ, unroll=16)
        def _(l):
            idx_l = zerov + l
            for c in range(DV):
                vp = pos_v[l, pl.ds(c * LANES, LANES)]
                idx_d = lane1 + c * LANES
                for s in range(SPC):
                    v = rows_v[buf, s * SEQ + l, pl.ds(c * LANES, LANES)]
                    plsc.store_scatter(outt_v.at[s], [idx_d, idx_l], v + vp)
        for s in range(SPC):
            pltpu.sync_copy(
                outt_v.at[s, :, pl.ds(0, SEQ)],
                out_hbm.at[seq_base + g * SPC + s],
            )

    for p in range(NBUF2 - 1):
        fire(p, p)

    @pl.loop(0, NCHUNK2, step=NBUF2)
    def _chunks(g0):
        for b in range(NBUF2):
            g = g0 + b
            fire(g + NBUF2 - 1, (b + NBUF2 - 1) % NBUF2)
            drain(b)
            process(g, b)


def kernel(x, table, pos):
    mesh = plsc.VectorSubcoreMesh(
        core_axis_name="c", subcore_axis_name="s",
        num_cores=NC, num_subcores=NS,
    )
    tblt = jnp.swapaxes(table, 0, 1)
    tail = jnp.swapaxes(table[NCHUNK1 * TC_:, :], 0, 1)

    packed = pl.kernel(
        _transpose_kernel,
        out_type=jax.ShapeDtypeStruct((V * PW1,), jnp.float32),
        mesh=mesh,
        scratch_types=[
            pltpu.VMEM((NBUF1, D, TC_), jnp.float32),
            pltpu.VMEM((D, VTAIL), jnp.float32),
            pltpu.VMEM((NWB * CHW,), jnp.float32),
        ] + [pltpu.SemaphoreType.DMA] * (NBUF1 + NWB),
        compiler_params=pltpu.CompilerParams(
            use_tc_tiling_on_sc=True, needs_layout_passes=False,
        ),
    )(tblt, tail)

    out = pl.kernel(
        _gather_kernel,
        out_type=jax.ShapeDtypeStruct((B, D, SEQ), jnp.float32),
        mesh=mesh,
        scratch_types=[
            pltpu.VMEM((SEQ_PER_W, SEQ), jnp.int32),
            pltpu.VMEM((SEQ, D), jnp.float32),
            pltpu.VMEM((NBUF2, SPC * SEQ, PW1), jnp.float32),
            pltpu.VMEM((SPC, D, PW2), jnp.float32),
            pltpu.SemaphoreType.DMA,
            pltpu.SemaphoreType.DMA,
            pltpu.SemaphoreType.DMA,
            pltpu.SemaphoreType.DMA,
        ],
        compiler_params=pltpu.CompilerParams(
            use_tc_tiling_on_sc=False, needs_layout_passes=False,
        ),
    )(x, packed.reshape(V, PW1), pos)

    return jnp.swapaxes(out, 1, 2)
